# baseline (speedup 1.0000x reference)
import functools

import jax
import jax.numpy as jnp
from jax import lax
from jax.experimental import pallas as pl
from jax.experimental.pallas import tpu as pltpu
from jax.experimental.pallas import tpu_sc as plsc

B, L, D, C = 4096, 200, 64, 7
NC, NS, LANES = 2, 16, 16
NW = NC * NS
ROWS_PER_W = B // NW
STAGE = 32
NCHUNK = 13
IDXN = 224
GMAX = NCHUNK * 16

_mesh = plsc.VectorSubcoreMesh(core_axis_name="c", subcore_axis_name="s")


def _pool_body(mids_hbm, table_hbm, out_hbm,
               mids_v, idx0_v, idx1_v, rows0_v, rows1_v, out_v, emb0_v,
               sem0, sem1):
    wid = lax.axis_index("s") * NC + lax.axis_index("c")
    w_base = wid * ROWS_PER_W
    iota = lax.iota(jnp.int32, LANES)

    def stage(s):
        sl = pl.ds(pl.multiple_of(w_base + s, STAGE), STAGE)
        pltpu.sync_copy(mids_hbm.at[sl, :], mids_v)

    def build_issue(r, idx_v, rows_ref, sem):
        rl = lax.rem(r, STAGE)
        off = jnp.int32(0)
        for j in range(NCHUNK):
            woff = j * 16 if j < NCHUNK - 1 else L - 16
            ids16 = mids_v[rl, pl.ds(woff, 16)]
            m = ids16 > 0
            if j == NCHUNK - 1:
                m = jnp.logical_and(m, iota >= NCHUNK * 16 - L)
            plsc.store_compressed(idx_v.at[pl.ds(off, 16)], ids16, mask=m)
            off = off + jnp.sum(m.astype(jnp.int32))
        nz = off
        idx_v[pl.ds(nz, 16)] = jnp.zeros((16,), jnp.int32)
        nc = (nz + 15) >> 4

        def issue(c, _):
            idx16 = idx_v[pl.ds(c * 16, 16)]
            pltpu.make_async_copy(
                table_hbm.at[idx16],
                rows_ref.at[pl.ds(c * 16, 16), :],
                sem).start()
            return _

        lax.fori_loop(0, nc, issue, jnp.int32(0))
        return nc

    def wait_acc(r, nc, rows_ref, sem):
        def drain(c, _):
            pltpu.make_async_copy(
                table_hbm.at[pl.ds(0, 16), :],
                rows_ref.at[pl.ds(c * 16, 16), :],
                sem).wait()
            return _

        lax.fori_loop(0, nc, drain, jnp.int32(0))

        def acc_chunk(c, acc):
            a0, a1, a2, a3 = acc
            for t in range(16):
                row = c * 16 + t
                a0 = a0 + rows_ref[row, pl.ds(0, 16)]
                a1 = a1 + rows_ref[row, pl.ds(16, 16)]
                a2 = a2 + rows_ref[row, pl.ds(32, 16)]
                a3 = a3 + rows_ref[row, pl.ds(48, 16)]
            return (a0, a1, a2, a3)

        z = jnp.zeros((LANES,), jnp.float32)
        acc = lax.fori_loop(0, nc, acc_chunk, (z, z, z, z))
        coef = (jnp.int32(L) - nc * 16).astype(jnp.float32)
        rl = lax.rem(r, STAGE)
        for d in range(4):
            out_v[rl, pl.ds(d * 16, 16)] = (
                acc[d] + coef * emb0_v[0, pl.ds(d * 16, 16)])

    def flush(r):
        s = r - (STAGE - 1)
        pltpu.sync_copy(
            out_v, out_hbm.at[pl.ds(pl.multiple_of(w_base + s, STAGE), STAGE), :])

    pltpu.sync_copy(table_hbm.at[pl.ds(0, 1), :], emb0_v)
    stage(jnp.int32(0))
    nc0 = build_issue(jnp.int32(0), idx0_v, rows0_v, sem0)

    def two_rows(rr, nc_even):
        r_odd = 2 * rr + 1
        r_even = 2 * rr + 2
        nc_odd = build_issue(r_odd, idx1_v, rows1_v, sem1)
        wait_acc(2 * rr, nc_even, rows0_v, sem0)

        @pl.when(lax.rem(r_even, STAGE) == 0)
        def _():
            stage(r_even)

        nc_even = build_issue(r_even, idx0_v, rows0_v, sem0)
        wait_acc(r_odd, nc_odd, rows1_v, sem1)

        @pl.when(lax.rem(r_odd, STAGE) == STAGE - 1)
        def _():
            flush(r_odd)

        return nc_even

    nc_last = lax.fori_loop(0, ROWS_PER_W // 2 - 1, two_rows, nc0)
    nc_odd = build_issue(jnp.int32(ROWS_PER_W - 1), idx1_v, rows1_v, sem1)
    wait_acc(jnp.int32(ROWS_PER_W - 2), nc_last, rows0_v, sem0)
    wait_acc(jnp.int32(ROWS_PER_W - 1), nc_odd, rows1_v, sem1)
    flush(jnp.int32(ROWS_PER_W - 1))


_pool = functools.partial(
    pl.kernel,
    out_type=jax.ShapeDtypeStruct((B, D), jnp.float32),
    mesh=_mesh,
    compiler_params=pltpu.CompilerParams(needs_layout_passes=False,
                                         use_tc_tiling_on_sc=False,
                                         disable_bounds_checks=True,
                                         skip_device_barrier=True),
    scratch_types=[
        pltpu.VMEM((STAGE, L), jnp.int32),
        pltpu.VMEM((IDXN,), jnp.int32),
        pltpu.VMEM((IDXN,), jnp.int32),
        pltpu.VMEM((GMAX, D), jnp.float32),
        pltpu.VMEM((GMAX, D), jnp.float32),
        pltpu.VMEM((STAGE, D), jnp.float32),
        pltpu.VMEM((1, D), jnp.float32),
        pltpu.SemaphoreType.DMA,
        pltpu.SemaphoreType.DMA,
    ],
)(_pool_body)


def _mlp_body(sums_ref, att_ref, drop_ref, lab_ref, W1_ref, b1_ref, W2_ref,
              b2_ref, W3_ref, b3_ref, loss_ref, logits_ref):
    nz = jnp.sum((att_ref[...] * drop_ref[...]).astype(jnp.float32), axis=1,
                 keepdims=True)
    x = sums_ref[...] / nz
    h = jnp.maximum(jnp.dot(x, W1_ref[...],
                            preferred_element_type=jnp.float32,
                            precision=lax.Precision.HIGHEST) + b1_ref[...], 0.0)
    h = jnp.maximum(jnp.dot(h, W2_ref[...],
                            preferred_element_type=jnp.float32,
                            precision=lax.Precision.HIGHEST) + b2_ref[...], 0.0)
    logits = jnp.dot(h, W3_ref[...],
                     preferred_element_type=jnp.float32,
                            precision=lax.Precision.HIGHEST) + b3_ref[...]
    col = lax.broadcasted_iota(jnp.int32, logits.shape, 1)
    valid = col < C
    neg = jnp.where(valid, logits, -1e30)
    m = jnp.max(neg, axis=1, keepdims=True)
    e = jnp.where(valid, jnp.exp(logits - m), 0.0)
    s = jnp.sum(e, axis=1, keepdims=True)
    logp = logits - m - jnp.log(s)
    pick = jnp.where(col == lab_ref[...], logp, 0.0)
    loss_ref[0, 0] = -jnp.sum(pick) / jnp.float32(B)
    logits_ref[...] = logits[:, :C]


def _mlp(sums, att, drop, lab2, W1, b1, W2, b2, W3p, b3p):
    return pl.pallas_call(
        _mlp_body,
        out_shape=(jax.ShapeDtypeStruct((1, 1), jnp.float32),
                   jax.ShapeDtypeStruct((B, C), jnp.float32)),
        out_specs=(pl.BlockSpec(memory_space=pltpu.SMEM),
                   pl.BlockSpec(memory_space=pltpu.VMEM)),
    )(sums, att, drop, lab2, W1, b1, W2, b2, W3p, b3p)


def kernel(input_ids, attention_masks, labels, dropout_mask, emb_table,
           W1, b1, W2, b2, W3, b3):
    att = attention_masks.astype(jnp.int32)
    drop = dropout_mask.astype(jnp.int32)
    mids = input_ids.astype(jnp.int32) * att * drop
    sums = _pool(mids, emb_table)

    W3p = jnp.zeros((D, 128), jnp.float32).at[:, :C].set(W3)
    b3p = jnp.zeros((128,), jnp.float32).at[:C].set(b3)
    lab2 = labels.astype(jnp.int32).reshape(B, 1)
    loss2, logits = _mlp(sums, att, drop, lab2, W1, b1.reshape(1, D),
                         W2, b2.reshape(1, D), W3p, b3p.reshape(1, 128))
    return loss2[0, 0], logits


# speedup vs baseline: 1.0014x; 1.0014x over previous
import functools

import jax
import jax.numpy as jnp
from jax import lax
from jax.experimental import pallas as pl
from jax.experimental.pallas import tpu as pltpu
from jax.experimental.pallas import tpu_sc as plsc

B, L, D, C = 4096, 200, 64, 7
NC, NS, LANES = 2, 16, 16
NW = NC * NS
ROWS_PER_W = B // NW
STAGE = 32
NCHUNK = 13
IDXN = 224
GMAX = NCHUNK * 16

_mesh = plsc.VectorSubcoreMesh(core_axis_name="c", subcore_axis_name="s")


def _pool_body(mids_hbm, table_hbm, out_hbm,
               mids_v, idx0_v, idx1_v, rows0_v, rows1_v, out_v, emb0_v,
               sem0, sem1):
    wid = lax.axis_index("s") * NC + lax.axis_index("c")
    w_base = wid * ROWS_PER_W
    iota = lax.iota(jnp.int32, LANES)

    def stage(s):
        sl = pl.ds(pl.multiple_of(w_base + s, STAGE), STAGE)
        pltpu.sync_copy(mids_hbm.at[sl, :], mids_v)

    def build_issue(r, idx_v, rows_ref, sem):
        rl = lax.rem(r, STAGE)
        off = jnp.int32(0)
        for j in range(NCHUNK):
            woff = j * 16 if j < NCHUNK - 1 else L - 16
            ids16 = mids_v[rl, pl.ds(woff, 16)]
            m = ids16 > 0
            if j == NCHUNK - 1:
                m = jnp.logical_and(m, iota >= NCHUNK * 16 - L)
            plsc.store_compressed(idx_v.at[pl.ds(off, 16)], ids16, mask=m)
            off = off + jnp.sum(m.astype(jnp.int32))
        nz = off
        idx_v[pl.ds(nz, 16)] = jnp.zeros((16,), jnp.int32)
        nc = (nz + 15) >> 4

        def issue(c, _):
            idx16 = idx_v[pl.ds(c * 16, 16)]
            pltpu.make_async_copy(
                table_hbm.at[idx16],
                rows_ref.at[pl.ds(c * 16, 16), :],
                sem).start()
            return _

        lax.fori_loop(0, nc, issue, jnp.int32(0))
        return nc

    def wait_acc(r, nc, rows_ref, sem):
        def drain(c, _):
            pltpu.make_async_copy(
                table_hbm.at[pl.ds(0, 16), :],
                rows_ref.at[pl.ds(c * 16, 16), :],
                sem).wait()
            return _

        lax.fori_loop(0, nc, drain, jnp.int32(0))

        def acc_chunk(c, acc):
            a0, a1, a2, a3 = acc
            for t in range(16):
                row = c * 16 + t
                a0 = a0 + rows_ref[row, pl.ds(0, 16)]
                a1 = a1 + rows_ref[row, pl.ds(16, 16)]
                a2 = a2 + rows_ref[row, pl.ds(32, 16)]
                a3 = a3 + rows_ref[row, pl.ds(48, 16)]
            return (a0, a1, a2, a3)

        z = jnp.zeros((LANES,), jnp.float32)
        acc = lax.fori_loop(0, nc, acc_chunk, (z, z, z, z))
        coef = (jnp.int32(L) - nc * 16).astype(jnp.float32)
        rl = lax.rem(r, STAGE)
        for d in range(4):
            out_v[rl, pl.ds(d * 16, 16)] = (
                acc[d] + coef * emb0_v[0, pl.ds(d * 16, 16)])

    def flush(r):
        s = r - (STAGE - 1)
        pltpu.sync_copy(
            out_v, out_hbm.at[pl.ds(pl.multiple_of(w_base + s, STAGE), STAGE), :])

    pltpu.sync_copy(table_hbm.at[pl.ds(0, 1), :], emb0_v)
    stage(jnp.int32(0))
    nc0 = build_issue(jnp.int32(0), idx0_v, rows0_v, sem0)

    def two_rows(rr, nc_even):
        r_odd = 2 * rr + 1
        r_even = 2 * rr + 2
        nc_odd = build_issue(r_odd, idx1_v, rows1_v, sem1)
        wait_acc(2 * rr, nc_even, rows0_v, sem0)

        @pl.when(lax.rem(r_even, STAGE) == 0)
        def _():
            stage(r_even)

        nc_even = build_issue(r_even, idx0_v, rows0_v, sem0)
        wait_acc(r_odd, nc_odd, rows1_v, sem1)

        @pl.when(lax.rem(r_odd, STAGE) == STAGE - 1)
        def _():
            flush(r_odd)

        return nc_even

    nc_last = lax.fori_loop(0, ROWS_PER_W // 2 - 1, two_rows, nc0)
    nc_odd = build_issue(jnp.int32(ROWS_PER_W - 1), idx1_v, rows1_v, sem1)
    wait_acc(jnp.int32(ROWS_PER_W - 2), nc_last, rows0_v, sem0)
    wait_acc(jnp.int32(ROWS_PER_W - 1), nc_odd, rows1_v, sem1)
    flush(jnp.int32(ROWS_PER_W - 1))


_pool = functools.partial(
    pl.kernel,
    out_type=jax.ShapeDtypeStruct((B, D), jnp.float32),
    mesh=_mesh,
    compiler_params=pltpu.CompilerParams(needs_layout_passes=False,
                                         use_tc_tiling_on_sc=False),
    scratch_types=[
        pltpu.VMEM((STAGE, L), jnp.int32),
        pltpu.VMEM((IDXN,), jnp.int32),
        pltpu.VMEM((IDXN,), jnp.int32),
        pltpu.VMEM((GMAX, D), jnp.float32),
        pltpu.VMEM((GMAX, D), jnp.float32),
        pltpu.VMEM((STAGE, D), jnp.float32),
        pltpu.VMEM((1, D), jnp.float32),
        pltpu.SemaphoreType.DMA,
        pltpu.SemaphoreType.DMA,
    ],
)(_pool_body)


def _mlp_body(sums_ref, att_ref, drop_ref, lab_ref, W1_ref, b1_ref, W2_ref,
              b2_ref, W3_ref, b3_ref, loss_ref, logits_ref):
    nz = jnp.sum((att_ref[...] * drop_ref[...]).astype(jnp.float32), axis=1,
                 keepdims=True)
    x = sums_ref[...] / nz
    h = jnp.maximum(jnp.dot(x, W1_ref[...],
                            preferred_element_type=jnp.float32,
                            precision=lax.Precision.HIGHEST) + b1_ref[...], 0.0)
    h = jnp.maximum(jnp.dot(h, W2_ref[...],
                            preferred_element_type=jnp.float32,
                            precision=lax.Precision.HIGHEST) + b2_ref[...], 0.0)
    logits = jnp.dot(h, W3_ref[...],
                     preferred_element_type=jnp.float32,
                            precision=lax.Precision.HIGHEST) + b3_ref[...]
    col = lax.broadcasted_iota(jnp.int32, logits.shape, 1)
    valid = col < C
    neg = jnp.where(valid, logits, -1e30)
    m = jnp.max(neg, axis=1, keepdims=True)
    e = jnp.where(valid, jnp.exp(logits - m), 0.0)
    s = jnp.sum(e, axis=1, keepdims=True)
    logp = logits - m - jnp.log(s)
    pick = jnp.where(col == lab_ref[...], logp, 0.0)
    loss_ref[0, 0] = -jnp.sum(pick) / jnp.float32(B)
    logits_ref[...] = logits[:, :C]


def _mlp(sums, att, drop, lab2, W1, b1, W2, b2, W3p, b3p):
    return pl.pallas_call(
        _mlp_body,
        out_shape=(jax.ShapeDtypeStruct((1, 1), jnp.float32),
                   jax.ShapeDtypeStruct((B, C), jnp.float32)),
        out_specs=(pl.BlockSpec(memory_space=pltpu.SMEM),
                   pl.BlockSpec(memory_space=pltpu.VMEM)),
    )(sums, att, drop, lab2, W1, b1, W2, b2, W3p, b3p)


def kernel(input_ids, attention_masks, labels, dropout_mask, emb_table,
           W1, b1, W2, b2, W3, b3):
    att = attention_masks.astype(jnp.int32)
    drop = dropout_mask.astype(jnp.int32)
    mids = input_ids.astype(jnp.int32) * att * drop
    sums = _pool(mids, emb_table)

    W3p = jnp.zeros((D, 128), jnp.float32).at[:, :C].set(W3)
    b3p = jnp.zeros((128,), jnp.float32).at[:C].set(b3)
    lab2 = labels.astype(jnp.int32).reshape(B, 1)
    loss2, logits = _mlp(sums, att, drop, lab2, W1, b1.reshape(1, D),
                         W2, b2.reshape(1, D), W3p, b3p.reshape(1, 128))
    return loss2[0, 0], logits
